# baseline (speedup 1.0000x reference)
import functools

import jax
import jax.numpy as jnp
from jax import lax
from jax.experimental import pallas as pl
from jax.experimental.pallas import tpu as pltpu
from jax.experimental.pallas import tpu_sc as plsc

_N = 10000
_D = 128
_E = 160000
_NC = 2
_NS = 16
_NW = _NC * _NS
_K = 88
_C = 57
_EPW = _K * _C
_G = 2
_NPAD = 10112
_RPW = _NPAD // _NS


def _sc_agg_body(x_hbm, src_hbm, dst_hbm, z_hbm,
                 out_sum, src_v, dst_v, rows_v, sem, acc_sh):
    cid = lax.axis_index("c")
    sid = lax.axis_index("s")
    wid = cid * _NS + sid
    pltpu.sync_copy(src_hbm.at[wid], src_v)
    pltpu.sync_copy(dst_hbm.at[wid], dst_v)
    zr = sid * _RPW
    pltpu.sync_copy(z_hbm.at[pl.ds(zr, _RPW)], acc_sh.at[pl.ds(zr, _RPW)])
    plsc.subcore_barrier()

    semg, sems = sem
    pltpu.async_copy(x_hbm.at[src_v.at[0]], rows_v.at[0], semg.at[0])
    pltpu.async_copy(x_hbm.at[src_v.at[1]], rows_v.at[1], semg.at[1])

    def step(c, carry):
        r = lax.rem(c, 3)
        pltpu.make_async_copy(x_hbm.at[src_v.at[c]], rows_v.at[r],
                              semg.at[r]).wait()

        @pl.when(c + 2 < _C)
        def _():
            r2 = lax.rem(c + 2, 3)

            @pl.when(c >= 1)
            def _():
                pltpu.make_async_copy(z_hbm.at[pl.ds(0, _K)],
                                      rows_v.at[r2], sems.at[r2]).wait()

            pltpu.async_copy(x_hbm.at[src_v.at[c + 2]], rows_v.at[r2],
                             semg.at[r2])

        pltpu.async_copy(rows_v.at[r], acc_sh.at[dst_v.at[c]],
                         sems.at[r], add=True)
        return carry

    lax.fori_loop(0, _C, step, 0)
    for i in range(3):
        c = _C - 1 - i
        pltpu.make_async_copy(z_hbm.at[pl.ds(0, _K)],
                              rows_v.at[lax.rem(c, 3)],
                              sems.at[lax.rem(c, 3)]).wait()
    plsc.subcore_barrier()
    pltpu.sync_copy(acc_sh.at[pl.ds(zr, _RPW)],
                    out_sum.at[cid, pl.ds(zr, _RPW)])


def _make_sc_agg():
    mesh = plsc.VectorSubcoreMesh(core_axis_name="c", subcore_axis_name="s")
    return pl.kernel(
        _sc_agg_body,
        mesh=mesh,
        out_type=jax.ShapeDtypeStruct((_NC, _NPAD, _D), jnp.float32),
        scratch_types=[
            pltpu.VMEM((_C, _K), jnp.int32),
            pltpu.VMEM((_C, _K), jnp.int32),
            pltpu.VMEM((3, _K, _D), jnp.float32),
            (pltpu.SemaphoreType.DMA((3,)), pltpu.SemaphoreType.DMA((3,))),
            pltpu.VMEM_SHARED((_NPAD, _D), jnp.float32),
        ],
    )


def _sc_deg_body(dst_hbm, zd_hbm, ones_hbm,
                 out_deg, dst_v, ones_v, deg_sh):
    cid = lax.axis_index("c")
    sid = lax.axis_index("s")
    wid = cid * _NS + sid
    pltpu.sync_copy(dst_hbm.at[wid], dst_v)
    pltpu.sync_copy(ones_hbm, ones_v)
    zr = sid * _RPW
    pltpu.sync_copy(zd_hbm.at[pl.ds(zr, _RPW)], deg_sh.at[pl.ds(zr, _RPW)])
    plsc.subcore_barrier()

    def step(j, carry):
        pltpu.sync_copy(ones_v, deg_sh.at[dst_v.at[j]], add=True)
        return carry

    lax.fori_loop(0, _C, step, 0)
    plsc.subcore_barrier()
    pltpu.sync_copy(deg_sh.at[pl.ds(zr, _RPW)],
                    out_deg.at[cid, pl.ds(zr, _RPW)])


def _make_sc_deg():
    mesh = plsc.VectorSubcoreMesh(core_axis_name="c", subcore_axis_name="s")
    return pl.kernel(
        _sc_deg_body,
        mesh=mesh,
        out_type=jax.ShapeDtypeStruct((_NC, _NPAD, _D), jnp.float32),
        scratch_types=[
            pltpu.VMEM((_C, _K), jnp.int32),
            pltpu.VMEM((_K, _D), jnp.float32),
            pltpu.VMEM_SHARED((_NPAD, _D), jnp.float32),
        ],
    )


def _prep_edges(ei):
    src = ei[0].reshape(_NW, _E // _NW)
    dst = ei[1].reshape(_NW, _E // _NW)
    padw = _EPW - _E // _NW
    src = jnp.pad(src, ((0, 0), (0, padw)), constant_values=0)
    dst = jnp.pad(dst, ((0, 0), (0, padw)), constant_values=_N)
    return src.reshape(_NW, _C, _K), dst.reshape(_NW, _C, _K)


_BN = 1000


def _layer_body(acc_ref, deg_ref, x_ref, wl_ref, bl_ref, wr_ref, o_ref):
    s = acc_ref[0] + acc_ref[1]
    deg = deg_ref[0, :, 0:1] + deg_ref[1, :, 0:1]
    agg = s * (1.0 / jnp.clip(deg, 1.0, None))
    h = jnp.dot(agg, wl_ref[...], preferred_element_type=jnp.float32)
    h = h + bl_ref[...]
    h = h + jnp.dot(x_ref[...], wr_ref[...], preferred_element_type=jnp.float32)
    o_ref[...] = jnp.maximum(h, 0.0)


def _tc_layer(acc, degp, x, Wl, bl, Wr):
    return pl.pallas_call(
        _layer_body,
        grid=(_N // _BN,),
        in_specs=[
            pl.BlockSpec((_NC, _BN, _D), lambda i: (0, i, 0)),
            pl.BlockSpec((_NC, _BN, _D), lambda i: (0, i, 0)),
            pl.BlockSpec((_BN, _D), lambda i: (i, 0)),
            pl.BlockSpec((_D, _D), lambda i: (0, 0)),
            pl.BlockSpec((1, _D), lambda i: (0, 0)),
            pl.BlockSpec((_D, _D), lambda i: (0, 0)),
        ],
        out_specs=pl.BlockSpec((_BN, _D), lambda i: (i, 0)),
        out_shape=jax.ShapeDtypeStruct((_N, _D), jnp.float32),
    )(acc, degp, x, Wl, bl.reshape(1, _D), Wr)


def _attn_body(h0_ref, h1_ref, h2_ref, h3_ref, w_ref, b_ref, a1_ref, b1_ref,
               a2_ref, b2_ref, r_ref, rb_ref, o_ref):
    hs = (h0_ref, h1_ref, h2_ref, h3_ref)
    feats = []
    logits = []
    for i in range(4):
        f = jnp.dot(hs[i][...], w_ref[i],
                    preferred_element_type=jnp.float32) + b_ref[i]
        hid = jnp.dot(f, a1_ref[...], preferred_element_type=jnp.float32)
        hid = jnp.maximum(hid + b1_ref[...], 0.0)
        lg = jnp.sum(hid * a2_ref[...], axis=1, keepdims=True) + b2_ref[...]
        feats.append(f)
        logits.append(lg)
    m = jnp.maximum(jnp.maximum(logits[0], logits[1]),
                    jnp.maximum(logits[2], logits[3]))
    es = [jnp.exp(l - m) for l in logits]
    tot = es[0] + es[1] + es[2] + es[3]
    weighted = sum(e * f for e, f in zip(es, feats)) / tot
    resid = jnp.dot(feats[0], r_ref[...],
                    preferred_element_type=jnp.float32) + rb_ref[...]
    o_ref[...] = jnp.maximum(weighted + resid, 0.0)


def _tc_attn(h_list, lin_params, attn_params):
    W = jnp.stack([w for w, _ in lin_params])
    b = jnp.stack([bb for _, bb in lin_params]).reshape(4, 1, 512)
    A1, b1, A2, b2, R, rb = attn_params
    return pl.pallas_call(
        _attn_body,
        grid=(_N // _BN,),
        in_specs=[
            pl.BlockSpec((_BN, _D), lambda i: (i, 0)),
            pl.BlockSpec((_BN, _D), lambda i: (i, 0)),
            pl.BlockSpec((_BN, _D), lambda i: (i, 0)),
            pl.BlockSpec((_BN, _D), lambda i: (i, 0)),
            pl.BlockSpec((4, _D, 512), lambda i: (0, 0, 0)),
            pl.BlockSpec((4, 1, 512), lambda i: (0, 0, 0)),
            pl.BlockSpec((512, _D), lambda i: (0, 0)),
            pl.BlockSpec((1, _D), lambda i: (0, 0)),
            pl.BlockSpec((1, _D), lambda i: (0, 0)),
            pl.BlockSpec((1, 1), lambda i: (0, 0)),
            pl.BlockSpec((512, 512), lambda i: (0, 0)),
            pl.BlockSpec((1, 512), lambda i: (0, 0)),
        ],
        out_specs=pl.BlockSpec((_BN, 512), lambda i: (i, 0)),
        out_shape=jax.ShapeDtypeStruct((_N, 512), jnp.float32),
    )(h_list[0], h_list[1], h_list[2], h_list[3], W, b, A1,
      b1.reshape(1, _D), A2.reshape(1, _D), b2.reshape(1, 1), R,
      rb.reshape(1, 512))


_FR = _N
_FCOLS = _N
_FBR = 80


def _fuse_body(fw_ref, cc_ref, cs_ref, o_ref):
    e0 = jnp.exp(fw_ref[0:1, 0:1])
    e1 = jnp.exp(fw_ref[0:1, 1:2])
    tot = e0 + e1
    o_ref[...] = (e0 / tot) * cc_ref[...] + (e1 / tot) * cs_ref[...]


def _tc_fuse(fusion_weight, C_c, C_s):
    cc = C_c
    cs = C_s
    out = pl.pallas_call(
        _fuse_body,
        grid=(_FR // _FBR,),
        in_specs=[
            pl.BlockSpec((1, 2), lambda i: (0, 0)),
            pl.BlockSpec((_FBR, _FCOLS), lambda i: (i, 0)),
            pl.BlockSpec((_FBR, _FCOLS), lambda i: (i, 0)),
        ],
        out_specs=pl.BlockSpec((_FBR, _FCOLS), lambda i: (i, 0)),
        out_shape=jax.ShapeDtypeStruct((_FR, _FCOLS), jnp.float32),
    )(fusion_weight.reshape(1, 2), cc, cs)
    return out


def kernel(x0, x1, x2, x3, x_content, sage_params, lin_params, attn_params,
           fusion_weight, C_c, C_s, edge_indices):
    agg = _make_sc_agg()
    deg_kern = _make_sc_deg()
    zeros = jnp.zeros((_NPAD, _D), jnp.float32)
    ones = jnp.ones((_K, _D), jnp.float32)

    edges = [_prep_edges(edge_indices[v]) for v in range(4)]
    degps = [deg_kern(edges[v][1], zeros, ones) for v in range(4)]
    hs = [x0, x1, x2, x3]
    for layer in range(3):
        accs = [agg(hs[v], edges[v][0], edges[v][1], zeros)
                for v in range(4)]
        for v in range(4):
            p = sage_params[v][layer]
            hs[v] = _tc_layer(accs[v], degps[v], hs[v], p[0], p[1], p[2])

    structure_features = _tc_attn(hs, lin_params, attn_params)
    fusion_expression = _tc_fuse(fusion_weight, C_c, C_s)
    return (fusion_expression, x_content, structure_features, C_c, C_s)


# speedup vs baseline: 1.1302x; 1.1302x over previous
import functools

import jax
import jax.numpy as jnp
from jax import lax
from jax.experimental import pallas as pl
from jax.experimental.pallas import tpu as pltpu
from jax.experimental.pallas import tpu_sc as plsc

_N = 10000
_D = 128
_E = 160000
_NC = 2
_NS = 16
_NW = _NC * _NS
_K = 88
_C = 57
_EPW = _K * _C
_G = 2
_NPAD = 10112
_RPW = _NPAD // _NS


def _sc_agg_body(x_hbm, src_hbm, dst_hbm, z_hbm,
                 out_sum, src_v, dst_v, rows_v, sem, acc_sh):
    cid = lax.axis_index("c")
    sid = lax.axis_index("s")
    wid = cid * _NS + sid
    pltpu.sync_copy(src_hbm.at[wid], src_v)
    pltpu.sync_copy(dst_hbm.at[wid], dst_v)
    zr = sid * _RPW
    pltpu.sync_copy(z_hbm.at[pl.ds(zr, _RPW)], acc_sh.at[pl.ds(zr, _RPW)])
    plsc.subcore_barrier()

    semg, sems = sem
    pltpu.async_copy(x_hbm.at[src_v.at[0]], rows_v.at[0], semg.at[0])
    pltpu.async_copy(x_hbm.at[src_v.at[1]], rows_v.at[1], semg.at[1])

    def step(c, carry):
        r = lax.rem(c, 3)
        pltpu.make_async_copy(x_hbm.at[src_v.at[c]], rows_v.at[r],
                              semg.at[r]).wait()

        @pl.when(c + 2 < _C)
        def _():
            r2 = lax.rem(c + 2, 3)

            @pl.when(c >= 1)
            def _():
                pltpu.make_async_copy(z_hbm.at[pl.ds(0, _K)],
                                      rows_v.at[r2], sems.at[r2]).wait()

            pltpu.async_copy(x_hbm.at[src_v.at[c + 2]], rows_v.at[r2],
                             semg.at[r2])

        pltpu.async_copy(rows_v.at[r], acc_sh.at[dst_v.at[c]],
                         sems.at[r], add=True)
        return carry

    lax.fori_loop(0, _C, step, 0)
    for i in range(3):
        c = _C - 1 - i
        pltpu.make_async_copy(z_hbm.at[pl.ds(0, _K)],
                              rows_v.at[lax.rem(c, 3)],
                              sems.at[lax.rem(c, 3)]).wait()
    plsc.subcore_barrier()
    pltpu.sync_copy(acc_sh.at[pl.ds(zr, _RPW)],
                    out_sum.at[cid, pl.ds(zr, _RPW)])


def _make_sc_agg():
    mesh = plsc.VectorSubcoreMesh(core_axis_name="c", subcore_axis_name="s")
    return pl.kernel(
        _sc_agg_body,
        mesh=mesh,
        out_type=jax.ShapeDtypeStruct((_NC, _NPAD, _D), jnp.float32),
        scratch_types=[
            pltpu.VMEM((_C, _K), jnp.int32),
            pltpu.VMEM((_C, _K), jnp.int32),
            pltpu.VMEM((3, _K, _D), jnp.float32),
            (pltpu.SemaphoreType.DMA((3,)), pltpu.SemaphoreType.DMA((3,))),
            pltpu.VMEM_SHARED((_NPAD, _D), jnp.float32),
        ],
    )


def _sc_deg_body(dst_hbm, zd_hbm, ones_hbm,
                 out_deg, dst_v, ones_v, deg_sh):
    cid = lax.axis_index("c")
    sid = lax.axis_index("s")
    wid = cid * _NS + sid
    pltpu.sync_copy(dst_hbm.at[wid], dst_v)
    pltpu.sync_copy(ones_hbm, ones_v)
    zr = sid * _RPW
    pltpu.sync_copy(zd_hbm.at[pl.ds(zr, _RPW)], deg_sh.at[pl.ds(zr, _RPW)])
    plsc.subcore_barrier()

    def step(j, carry):
        pltpu.sync_copy(ones_v, deg_sh.at[dst_v.at[j]], add=True)
        return carry

    lax.fori_loop(0, _C, step, 0)
    plsc.subcore_barrier()
    pltpu.sync_copy(deg_sh.at[pl.ds(zr, _RPW)],
                    out_deg.at[cid, pl.ds(zr, _RPW)])


def _make_sc_deg():
    mesh = plsc.VectorSubcoreMesh(core_axis_name="c", subcore_axis_name="s")
    return pl.kernel(
        _sc_deg_body,
        mesh=mesh,
        out_type=jax.ShapeDtypeStruct((_NC, _NPAD, _D), jnp.float32),
        scratch_types=[
            pltpu.VMEM((_C, _K), jnp.int32),
            pltpu.VMEM((_K, _D), jnp.float32),
            pltpu.VMEM_SHARED((_NPAD, _D), jnp.float32),
        ],
    )


def _prep_edges(ei):
    src = ei[0].reshape(_NW, _E // _NW)
    dst = ei[1].reshape(_NW, _E // _NW)
    padw = _EPW - _E // _NW
    src = jnp.pad(src, ((0, 0), (0, padw)), constant_values=0)
    dst = jnp.pad(dst, ((0, 0), (0, padw)), constant_values=_N)
    return src.reshape(_NW, _C, _K), dst.reshape(_NW, _C, _K)


_BN = 1000


def _layer_body(acc_ref, deg_ref, x_ref, wl_ref, bl_ref, wr_ref, o_ref):
    s = acc_ref[0] + acc_ref[1]
    deg = deg_ref[0, :, 0:1] + deg_ref[1, :, 0:1]
    agg = s * (1.0 / jnp.clip(deg, 1.0, None))
    h = jnp.dot(agg, wl_ref[...], preferred_element_type=jnp.float32)
    h = h + bl_ref[...]
    h = h + jnp.dot(x_ref[...], wr_ref[...], preferred_element_type=jnp.float32)
    o_ref[...] = jnp.maximum(h, 0.0)


def _tc_layer(acc, degp, x, Wl, bl, Wr):
    return pl.pallas_call(
        _layer_body,
        grid=(_N // _BN,),
        in_specs=[
            pl.BlockSpec((_NC, _BN, _D), lambda i: (0, i, 0)),
            pl.BlockSpec((_NC, _BN, _D), lambda i: (0, i, 0)),
            pl.BlockSpec((_BN, _D), lambda i: (i, 0)),
            pl.BlockSpec((_D, _D), lambda i: (0, 0)),
            pl.BlockSpec((1, _D), lambda i: (0, 0)),
            pl.BlockSpec((_D, _D), lambda i: (0, 0)),
        ],
        out_specs=pl.BlockSpec((_BN, _D), lambda i: (i, 0)),
        out_shape=jax.ShapeDtypeStruct((_N, _D), jnp.float32),
    )(acc, degp, x, Wl, bl.reshape(1, _D), Wr)


def _attn_body(h0_ref, h1_ref, h2_ref, h3_ref, w_ref, b_ref, a1_ref, b1_ref,
               a2_ref, b2_ref, r_ref, rb_ref, o_ref):
    hs = (h0_ref, h1_ref, h2_ref, h3_ref)
    feats = []
    logits = []
    for i in range(4):
        f = jnp.dot(hs[i][...], w_ref[i],
                    preferred_element_type=jnp.float32) + b_ref[i]
        hid = jnp.dot(f, a1_ref[...], preferred_element_type=jnp.float32)
        hid = jnp.maximum(hid + b1_ref[...], 0.0)
        lg = jnp.sum(hid * a2_ref[...], axis=1, keepdims=True) + b2_ref[...]
        feats.append(f)
        logits.append(lg)
    m = jnp.maximum(jnp.maximum(logits[0], logits[1]),
                    jnp.maximum(logits[2], logits[3]))
    es = [jnp.exp(l - m) for l in logits]
    tot = es[0] + es[1] + es[2] + es[3]
    weighted = sum(e * f for e, f in zip(es, feats)) / tot
    resid = jnp.dot(feats[0], r_ref[...],
                    preferred_element_type=jnp.float32) + rb_ref[...]
    o_ref[...] = jnp.maximum(weighted + resid, 0.0)


def _tc_attn(h_list, lin_params, attn_params):
    W = jnp.stack([w for w, _ in lin_params])
    b = jnp.stack([bb for _, bb in lin_params]).reshape(4, 1, 512)
    A1, b1, A2, b2, R, rb = attn_params
    return pl.pallas_call(
        _attn_body,
        grid=(_N // _BN,),
        in_specs=[
            pl.BlockSpec((_BN, _D), lambda i: (i, 0)),
            pl.BlockSpec((_BN, _D), lambda i: (i, 0)),
            pl.BlockSpec((_BN, _D), lambda i: (i, 0)),
            pl.BlockSpec((_BN, _D), lambda i: (i, 0)),
            pl.BlockSpec((4, _D, 512), lambda i: (0, 0, 0)),
            pl.BlockSpec((4, 1, 512), lambda i: (0, 0, 0)),
            pl.BlockSpec((512, _D), lambda i: (0, 0)),
            pl.BlockSpec((1, _D), lambda i: (0, 0)),
            pl.BlockSpec((1, _D), lambda i: (0, 0)),
            pl.BlockSpec((1, 1), lambda i: (0, 0)),
            pl.BlockSpec((512, 512), lambda i: (0, 0)),
            pl.BlockSpec((1, 512), lambda i: (0, 0)),
        ],
        out_specs=pl.BlockSpec((_BN, 512), lambda i: (i, 0)),
        out_shape=jax.ShapeDtypeStruct((_N, 512), jnp.float32),
    )(h_list[0], h_list[1], h_list[2], h_list[3], W, b, A1,
      b1.reshape(1, _D), A2.reshape(1, _D), b2.reshape(1, 1), R,
      rb.reshape(1, 512))


_FR = _N
_FCOLS = _N
_FBR = 80


def _fuse_body(fw_ref, cc_ref, cs_ref, o_ref, occ_ref, ocs_ref):
    e0 = jnp.exp(fw_ref[0:1, 0:1])
    e1 = jnp.exp(fw_ref[0:1, 1:2])
    tot = e0 + e1
    cc = cc_ref[...]
    cs = cs_ref[...]
    o_ref[...] = (e0 / tot) * cc + (e1 / tot) * cs
    occ_ref[...] = cc
    ocs_ref[...] = cs


def _tc_fuse(fusion_weight, C_c, C_s):
    cc = C_c
    cs = C_s
    out = pl.pallas_call(
        _fuse_body,
        grid=(_FR // _FBR,),
        in_specs=[
            pl.BlockSpec((1, 2), lambda i: (0, 0)),
            pl.BlockSpec((_FBR, _FCOLS), lambda i: (i, 0)),
            pl.BlockSpec((_FBR, _FCOLS), lambda i: (i, 0)),
        ],
        out_specs=[pl.BlockSpec((_FBR, _FCOLS), lambda i: (i, 0))] * 3,
        out_shape=[jax.ShapeDtypeStruct((_FR, _FCOLS), jnp.float32)] * 3,
    )(fusion_weight.reshape(1, 2), cc, cs)
    return out


def kernel(x0, x1, x2, x3, x_content, sage_params, lin_params, attn_params,
           fusion_weight, C_c, C_s, edge_indices):
    agg = _make_sc_agg()
    deg_kern = _make_sc_deg()
    zeros = jnp.zeros((_NPAD, _D), jnp.float32)
    ones = jnp.ones((_K, _D), jnp.float32)

    edges = [_prep_edges(edge_indices[v]) for v in range(4)]
    degps = [deg_kern(edges[v][1], zeros, ones) for v in range(4)]
    hs = [x0, x1, x2, x3]
    for layer in range(3):
        accs = [agg(hs[v], edges[v][0], edges[v][1], zeros)
                for v in range(4)]
        for v in range(4):
            p = sage_params[v][layer]
            hs[v] = _tc_layer(accs[v], degps[v], hs[v], p[0], p[1], p[2])

    structure_features = _tc_attn(hs, lin_params, attn_params)
    fusion_expression, cc_out, cs_out = _tc_fuse(fusion_weight, C_c, C_s)
    return (fusion_expression, x_content, structure_features, cc_out, cs_out)
